# baseline (speedup 1.0000x reference)
import jax
import jax.numpy as jnp
from jax.experimental import pallas as pl
from jax.experimental.pallas import tpu as pltpu


def _round_up(v: int, m: int) -> int:
    return ((v + m - 1) // m) * m


def _ffq_bf16(x):
    bi = jax.lax.bitcast_convert_type(x, jnp.int32)
    absbits = bi & jnp.int32(0x7FFFFFFF)
    signbit = bi & jnp.int32(-2147483648)
    trunc = jax.lax.bitcast_convert_type(bi & jnp.int32(-8192), jnp.float32)
    e_field = jax.lax.shift_right_logical(absbits, 23)
    sfac_bits = (jnp.minimum(143 - e_field, 0) + 127) << 23
    sfac = jax.lax.bitcast_convert_type(sfac_bits, jnp.float32)
    r_ge1 = trunc * sfac
    a = jax.lax.bitcast_convert_type(absbits, jnp.float32)
    f0 = jnp.floor(a * 1024.0) * jnp.float32(2.0 ** -10)
    fixed = jax.lax.bitcast_convert_type(
        jax.lax.bitcast_convert_type(f0, jnp.int32) | signbit, jnp.float32
    )
    q = jnp.where(absbits < jnp.int32(0x3F800000), fixed, r_ge1)
    return q.astype(jnp.bfloat16)


def _quant_bf16_kernel(x_ref, o_ref):
    o_ref[...] = _ffq_bf16(x_ref[...])


def _quantize_to_bf16(x, block_r):
    R, C = x.shape
    return pl.pallas_call(
        _quant_bf16_kernel,
        out_shape=jax.ShapeDtypeStruct((R, C), jnp.bfloat16),
        grid=(R // block_r,),
        in_specs=[pl.BlockSpec((block_r, C), lambda i: (i, 0))],
        out_specs=pl.BlockSpec((block_r, C), lambda i: (i, 0)),
        compiler_params=pltpu.CompilerParams(
            dimension_semantics=("parallel",),
        ),
    )(x)


def _fused_mm_kernel(x_ref, w_ref, b_ref, o_ref, xq_ref, acc_ref, *, bk):
    n = pl.program_id(1)
    k = pl.program_id(2)
    nk = pl.num_programs(2)

    @pl.when(n == 0)
    def _():
        xq_ref[:, pl.ds(k * bk, bk)] = _ffq_bf16(x_ref[...])

    wq = _ffq_bf16(w_ref[...])
    partial = jax.lax.dot_general(
        xq_ref[:, pl.ds(k * bk, bk)],
        wq,
        dimension_numbers=(((1,), (1,)), ((), ())),
        preferred_element_type=jnp.float32,
    )

    @pl.when(k == 0)
    def _():
        acc_ref[...] = partial

    @pl.when(k > 0)
    def _():
        acc_ref[...] += partial

    @pl.when(k == nk - 1)
    def _():
        o_ref[...] = acc_ref[...] + b_ref[...]


def _fused_matmul_bias(x, w, bias2d, bm, bn, bk):
    import functools
    M, K = x.shape
    N, _ = w.shape
    nk = K // bk
    return pl.pallas_call(
        functools.partial(_fused_mm_kernel, bk=bk),
        out_shape=jax.ShapeDtypeStruct((M, N), jnp.float32),
        grid=(M // bm, N // bn, nk),
        in_specs=[
            pl.BlockSpec((bm, bk),
                         lambda i, n, k: (i, jnp.where(n == 0, k, nk - 1))),
            pl.BlockSpec((bn, bk), lambda i, n, k: (n, k)),
            pl.BlockSpec((1, bn), lambda i, n, k: (0, n)),
        ],
        out_specs=pl.BlockSpec((bm, bn), lambda i, n, k: (i, n)),
        scratch_shapes=[
            pltpu.VMEM((bm, K), jnp.bfloat16),
            pltpu.VMEM((bm, bn), jnp.float32),
        ],
        compiler_params=pltpu.CompilerParams(
            dimension_semantics=("parallel", "arbitrary", "arbitrary"),
            vmem_limit_bytes=100 * 1024 * 1024,
        ),
    )(x, w, bias2d)


def kernel(x, weight, bias):
    M, K = x.shape
    N, Kw = weight.shape
    assert K == Kw

    x = x.astype(jnp.float32)
    weight = weight.astype(jnp.float32)
    if bias is None:
        bias = jnp.zeros((N,), jnp.float32)
    bias = bias.astype(jnp.float32)

    bm = min(2048, _round_up(M, 8))
    bn = min(1024, _round_up(N, 128))
    bk = min(512, _round_up(K, 128))
    M_pad = _round_up(M, bm)
    N_pad = _round_up(N, bn)
    K_pad = _round_up(K, bk)
    if (M_pad, K_pad) != (M, K):
        x = jnp.pad(x, ((0, M_pad - M), (0, K_pad - K)))
    if (N_pad, K_pad) != (N, K):
        weight = jnp.pad(weight, ((0, N_pad - N), (0, K_pad - K)))
    if N_pad != N:
        bias = jnp.pad(bias, (0, N_pad - N))
    bias2d = bias.reshape(1, N_pad)

    out = _fused_matmul_bias(x, weight, bias2d, bm, bn, bk)
    return out[:M, :N]


# speedup vs baseline: 1.2251x; 1.2251x over previous
import jax
import jax.numpy as jnp
from jax.experimental import pallas as pl
from jax.experimental.pallas import tpu as pltpu


def _round_up(v: int, m: int) -> int:
    return ((v + m - 1) // m) * m


def _ffq_bf16(x):
    bi = jax.lax.bitcast_convert_type(x, jnp.int32)
    absbits = bi & jnp.int32(0x7FFFFFFF)
    signbit = bi & jnp.int32(-2147483648)
    trunc = jax.lax.bitcast_convert_type(bi & jnp.int32(-8192), jnp.float32)
    e_field = jax.lax.shift_right_logical(absbits, 23)
    sfac_bits = (jnp.minimum(143 - e_field, 0) + 127) << 23
    sfac = jax.lax.bitcast_convert_type(sfac_bits, jnp.float32)
    r_ge1 = trunc * sfac
    a = jax.lax.bitcast_convert_type(absbits, jnp.float32)
    f0 = jnp.floor(a * 1024.0) * jnp.float32(2.0 ** -10)
    fixed = jax.lax.bitcast_convert_type(
        jax.lax.bitcast_convert_type(f0, jnp.int32) | signbit, jnp.float32
    )
    q = jnp.where(absbits < jnp.int32(0x3F800000), fixed, r_ge1)
    return q.astype(jnp.bfloat16)


def _fused_mm_kernel(x_ref, w_ref, b_ref, o_ref, xq_ref):
    j = pl.program_id(1)

    @pl.when(j == 0)
    def _():
        xq_ref[...] = _ffq_bf16(x_ref[...])

    h = w_ref.shape[0] // 2
    xq = xq_ref[...]
    dn = (((1,), (1,)), ((), ()))
    wq0 = _ffq_bf16(w_ref[:h, :])
    acc0 = jax.lax.dot_general(xq, wq0, dimension_numbers=dn,
                               preferred_element_type=jnp.float32)
    wq1 = _ffq_bf16(w_ref[h:, :])
    acc1 = jax.lax.dot_general(xq, wq1, dimension_numbers=dn,
                               preferred_element_type=jnp.float32)
    o_ref[:, :h] = acc0 + b_ref[:, :h]
    o_ref[:, h:] = acc1 + b_ref[:, h:]


def _fused_matmul_bias(x, w, bias2d, bm, bn):
    M, K = x.shape
    N, _ = w.shape
    return pl.pallas_call(
        _fused_mm_kernel,
        out_shape=jax.ShapeDtypeStruct((M, N), jnp.float32),
        grid=(M // bm, N // bn),
        in_specs=[
            pl.BlockSpec((bm, K), lambda i, j: (i, 0)),
            pl.BlockSpec((bn, K), lambda i, j: (j, 0)),
            pl.BlockSpec((1, bn), lambda i, j: (0, j)),
        ],
        out_specs=pl.BlockSpec((bm, bn), lambda i, j: (i, j)),
        scratch_shapes=[pltpu.VMEM((bm, K), jnp.bfloat16)],
        compiler_params=pltpu.CompilerParams(
            dimension_semantics=("parallel", "arbitrary"),
            vmem_limit_bytes=100 * 1024 * 1024,
        ),
    )(x, w, bias2d)


def kernel(x, weight, bias):
    M, K = x.shape
    N, Kw = weight.shape
    assert K == Kw

    x = x.astype(jnp.float32)
    weight = weight.astype(jnp.float32)
    if bias is None:
        bias = jnp.zeros((N,), jnp.float32)
    bias = bias.astype(jnp.float32)

    bm = min(1024, _round_up(M, 8))
    bn = min(512, _round_up(N, 256))
    M_pad = _round_up(M, bm)
    N_pad = _round_up(N, bn)
    K_pad = _round_up(K, 128)
    if (M_pad, K_pad) != (M, K):
        x = jnp.pad(x, ((0, M_pad - M), (0, K_pad - K)))
    if (N_pad, K_pad) != (N, K):
        weight = jnp.pad(weight, ((0, N_pad - N), (0, K_pad - K)))
    if N_pad != N:
        bias = jnp.pad(bias, (0, N_pad - N))
    bias2d = bias.reshape(1, N_pad)

    out = _fused_matmul_bias(x, weight, bias2d, bm, bn)
    return out[:M, :N]


# speedup vs baseline: 1.2801x; 1.0449x over previous
import jax
import jax.numpy as jnp
from jax.experimental import pallas as pl
from jax.experimental.pallas import tpu as pltpu


def _round_up(v: int, m: int) -> int:
    return ((v + m - 1) // m) * m


def _ffq_bf16(x):
    bi = jax.lax.bitcast_convert_type(x, jnp.int32)
    absbits = bi & jnp.int32(0x7FFFFFFF)
    signbit = bi & jnp.int32(-2147483648)
    trunc = jax.lax.bitcast_convert_type(bi & jnp.int32(-8192), jnp.float32)
    e_field = jax.lax.shift_right_logical(absbits, 23)
    sfac_bits = (jnp.minimum(143 - e_field, 0) + 127) << 23
    sfac = jax.lax.bitcast_convert_type(sfac_bits, jnp.float32)
    r_ge1 = trunc * sfac
    a = jax.lax.bitcast_convert_type(absbits, jnp.float32)
    f0 = jnp.floor(a * 1024.0) * jnp.float32(2.0 ** -10)
    fixed = jax.lax.bitcast_convert_type(
        jax.lax.bitcast_convert_type(f0, jnp.int32) | signbit, jnp.float32
    )
    q = jnp.where(absbits < jnp.int32(0x3F800000), fixed, r_ge1)
    return q.astype(jnp.bfloat16)


def _fused_mm_kernel(x_ref, w_ref, b_ref, o_ref, xq_ref):
    j = pl.program_id(1)

    @pl.when(j == 0)
    def _():
        xq_ref[...] = _ffq_bf16(x_ref[...])

    h = w_ref.shape[0] // 2
    xq = xq_ref[...]
    dn = (((1,), (1,)), ((), ()))
    wq0 = _ffq_bf16(w_ref[:h, :])
    acc0 = jax.lax.dot_general(xq, wq0, dimension_numbers=dn,
                               preferred_element_type=jnp.float32)
    wq1 = _ffq_bf16(w_ref[h:, :])
    acc1 = jax.lax.dot_general(xq, wq1, dimension_numbers=dn,
                               preferred_element_type=jnp.float32)
    o_ref[:, :h] = acc0 + b_ref[:, :h]
    o_ref[:, h:] = acc1 + b_ref[:, h:]


def _fused_matmul_bias(x, w, bias2d, bm, bn):
    M, K = x.shape
    N, _ = w.shape
    nj = N // bn

    def _j2(i, j):
        return jnp.where((i % 2) == 1, nj - 1 - j, j)

    return pl.pallas_call(
        _fused_mm_kernel,
        out_shape=jax.ShapeDtypeStruct((M, N), jnp.float32),
        grid=(M // bm, nj),
        in_specs=[
            pl.BlockSpec((bm, K), lambda i, j: (i, 0)),
            pl.BlockSpec((bn, K), lambda i, j: (_j2(i, j), 0)),
            pl.BlockSpec((1, bn), lambda i, j: (0, _j2(i, j))),
        ],
        out_specs=pl.BlockSpec((bm, bn), lambda i, j: (i, _j2(i, j))),
        scratch_shapes=[pltpu.VMEM((bm, K), jnp.bfloat16)],
        compiler_params=pltpu.CompilerParams(
            dimension_semantics=("parallel", "arbitrary"),
            vmem_limit_bytes=100 * 1024 * 1024,
        ),
    )(x, w, bias2d)


def kernel(x, weight, bias):
    M, K = x.shape
    N, Kw = weight.shape
    assert K == Kw

    x = x.astype(jnp.float32)
    weight = weight.astype(jnp.float32)
    if bias is None:
        bias = jnp.zeros((N,), jnp.float32)
    bias = bias.astype(jnp.float32)

    bm = min(1024, _round_up(M, 8))
    bn = min(512, _round_up(N, 256))
    M_pad = _round_up(M, bm)
    N_pad = _round_up(N, bn)
    K_pad = _round_up(K, 128)
    if (M_pad, K_pad) != (M, K):
        x = jnp.pad(x, ((0, M_pad - M), (0, K_pad - K)))
    if (N_pad, K_pad) != (N, K):
        weight = jnp.pad(weight, ((0, N_pad - N), (0, K_pad - K)))
    if N_pad != N:
        bias = jnp.pad(bias, (0, N_pad - N))
    bias2d = bias.reshape(1, N_pad)

    out = _fused_matmul_bias(x, weight, bias2d, bm, bn)
    return out[:M, :N]
